# baseline (speedup 1.0000x reference)
import functools

import jax
import jax.numpy as jnp
from jax import lax
from jax.experimental import pallas as pl
from jax.experimental.pallas import tpu as pltpu
from jax.experimental.pallas import tpu_sc as plsc

D = 64
D2 = 2 * D
B = 16384
LMBDA = 0.0001

NC = 2
NS = 16
NW = NC * NS
BPW = B // NW

IC = 128
CHUNKS = BPW // IC


@functools.cache
def _sc_gather():
    mesh = plsc.VectorSubcoreMesh(core_axis_name="c", subcore_axis_name="s")
    out_t = [jax.ShapeDtypeStruct((B, D2), jnp.float32)] * 6
    scratch = [
        pltpu.VMEM((CHUNKS, IC), jnp.int32),
        pltpu.VMEM((BPW, D2), jnp.float32),
        pltpu.SemaphoreType.DMA,
    ]

    @functools.partial(pl.kernel, mesh=mesh, out_type=out_t,
                       scratch_types=scratch)
    def k(ph, pt, pr, nh, nt, nr, ent, rel,
          o_ph, o_pt, o_pr, o_nh, o_nt, o_nr,
          idx_v, rows, sem):
        wid = lax.axis_index("s") * NC + lax.axis_index("c")
        base = wid * BPW
        wrow = wid * CHUNKS
        pairs = [(ph, ent, o_ph), (pt, ent, o_pt), (pr, rel, o_pr),
                 (nh, ent, o_nh), (nt, ent, o_nt), (nr, rel, o_nr)]
        for idx_hbm, table, out in pairs:
            pltpu.sync_copy(idx_hbm.at[pl.ds(wrow, CHUNKS)], idx_v)
            handles = [
                pltpu.async_copy(table.at[idx_v.at[j]],
                                 rows.at[pl.ds(j * IC, IC)], sem)
                for j in range(CHUNKS)
            ]
            for h in handles:
                h.wait()
            pltpu.sync_copy(rows, out.at[pl.ds(base, BPW)])

    return k


def _tc_loss(gathered, parities):
    BLK = 2048

    def body(ph_ref, pt_ref, pr_ref, nh_ref, nt_ref, nr_ref,
             hp_h, hp_t, hp_r, hn_h, hn_t, hn_r, out_ref):
        @pl.when(pl.program_id(0) == 0)
        def _():
            out_ref[0, 0] = 0.0

        def pick(ref, par):
            two = ref[...]
            return jnp.where(par[...] == 1, two[:, D:], two[:, :D])

        phv = pick(ph_ref, hp_h)
        ptv = pick(pt_ref, hp_t)
        prv = pick(pr_ref, hp_r)
        nhv = pick(nh_ref, hn_h)
        ntv = pick(nt_ref, hn_t)
        nrv = pick(nr_ref, hn_r)
        p = jnp.sum(phv * prv * ptv, axis=-1)
        n = jnp.sum(nhv * nrv * ntv, axis=-1)
        lf = jnp.sum(jax.nn.softplus(-p) + jax.nn.softplus(n))
        rg = jnp.sum(phv * phv + ptv * ptv + prv * prv
                     + nhv * nhv + ntv * ntv + nrv * nrv)
        out_ref[0, 0] += lf + LMBDA * rg

    rspec = pl.BlockSpec((BLK, D2), lambda i: (i, 0))
    pspec = pl.BlockSpec((BLK, 1), lambda i: (i, 0))
    out = pl.pallas_call(
        body,
        grid=(B // BLK,),
        in_specs=[rspec] * 6 + [pspec] * 6,
        out_specs=pl.BlockSpec(memory_space=pltpu.SMEM),
        out_shape=jax.ShapeDtypeStruct((1, 1), jnp.float32),
    )(*gathered, *parities)
    return out[0, 0]


def kernel(pos_h, pos_t, pos_r, neg_h, neg_t, neg_r,
           ent_embeddings, rel_embeddings):
    idxs = [x.astype(jnp.int32) for x in
            (pos_h, pos_t, pos_r, neg_h, neg_t, neg_r)]
    phys = [(x >> 1).reshape(B // IC, IC) for x in idxs]
    pars = [(x & 1).reshape(B, 1) for x in idxs]
    ent2 = ent_embeddings.reshape(-1, D2)
    rel2 = rel_embeddings.reshape(-1, D2)
    gathered = _sc_gather()(*phys, ent2, rel2)
    return _tc_loss(gathered, pars)
